# baseline (speedup 1.0000x reference)
import functools

import jax
import jax.numpy as jnp
from jax import lax
from jax.experimental import pallas as pl
from jax.experimental.pallas import tpu as pltpu
from jax.experimental.pallas import tpu_sc as plsc

BATCH = 16384
EMBED_DIM = 64
NUM_CORES = 2
NUM_SUBCORES = 16
NUM_WORKERS = NUM_CORES * NUM_SUBCORES
ROWS_PER_WORKER = BATCH // NUM_WORKERS
LANES = 16
NUM_PASSES = 2
HALF = ROWS_PER_WORKER // NUM_PASSES
HGRP = HALF // LANES


def _body(a_hbm, b_hbm, emb_hbm, out_hbm,
          a_idx, b_idx, a_rows, b_rows, out_v, sem):
    wid = lax.axis_index("s") * NUM_CORES + lax.axis_index("c")
    base = wid * ROWS_PER_WORKER

    pltpu.sync_copy(a_hbm.at[pl.ds(base, ROWS_PER_WORKER)], a_idx)
    pltpu.sync_copy(b_hbm.at[pl.ds(base, ROWS_PER_WORKER)], b_idx)

    lane = lax.iota(jnp.int32, LANES)

    for p in range(NUM_PASSES):
        def fetch(i, carry):
            av = a_idx[pl.ds(p * HALF + i * LANES, LANES)]
            bv = b_idx[pl.ds(p * HALF + i * LANES, LANES)]
            for j in range(LANES):
                row = i * LANES + j
                pltpu.async_copy(emb_hbm.at[av[j]], a_rows.at[row], sem)
                pltpu.async_copy(emb_hbm.at[bv[j]], b_rows.at[row], sem)
            return carry

        lax.fori_loop(0, HALF // LANES, fetch, 0)
        pltpu.make_async_copy(emb_hbm.at[pl.ds(0, HALF)], a_rows, sem).wait()
        pltpu.make_async_copy(emb_hbm.at[pl.ds(0, HALF)], b_rows, sem).wait()

        def group(g, carry):
            sums = jnp.zeros((LANES,), jnp.float32)
            for r16 in range(LANES):
                row = g * LANES + r16
                acc = jnp.zeros((LANES,), jnp.float32)
                for k in range(EMBED_DIM // LANES):
                    sl = pl.ds(k * LANES, LANES)
                    acc = acc + a_rows[row, sl] * b_rows[row, sl]
                sums = jnp.where(lane == r16, jnp.sum(acc), sums)
            y = 1.0 / (1.0 + jnp.exp(-sums))
            out_v[pl.ds(p * HALF + g * LANES, LANES)] = y
            return carry

        lax.fori_loop(0, HGRP, group, 0)

    pltpu.sync_copy(out_v, out_hbm.at[pl.ds(base, ROWS_PER_WORKER)])


@jax.jit
def _cooc_dssm(a_nid, b_nid, nid_emb):
    mesh = plsc.VectorSubcoreMesh(core_axis_name="c", subcore_axis_name="s")
    kern = functools.partial(
        pl.kernel,
        mesh=mesh,
        out_type=jax.ShapeDtypeStruct((BATCH,), jnp.float32),
        scratch_types=[
            pltpu.VMEM((ROWS_PER_WORKER,), jnp.int32),
            pltpu.VMEM((ROWS_PER_WORKER,), jnp.int32),
            pltpu.VMEM((HALF, EMBED_DIM), jnp.float32),
            pltpu.VMEM((HALF, EMBED_DIM), jnp.float32),
            pltpu.VMEM((ROWS_PER_WORKER,), jnp.float32),
            pltpu.SemaphoreType.DMA,
        ],
        compiler_params=pltpu.CompilerParams(
            needs_layout_passes=False, use_tc_tiling_on_sc=True),
    )(_body)
    return kern(a_nid, b_nid, nid_emb)


def kernel(a_nid, b_nid, nid_emb):
    return _cooc_dssm(a_nid.astype(jnp.int32), b_nid.astype(jnp.int32),
                      nid_emb)


# speedup vs baseline: 1.4890x; 1.4890x over previous
import functools

import jax
import jax.numpy as jnp
from jax import lax
from jax.experimental import pallas as pl
from jax.experimental.pallas import tpu as pltpu
from jax.experimental.pallas import tpu_sc as plsc

MOVIES = 1000000
BATCH = 16384
EMBED_DIM = 64
SUBLANES = 8
TILES = MOVIES // SUBLANES
NUM_CORES = 2
NUM_SUBCORES = 16
NUM_WORKERS = NUM_CORES * NUM_SUBCORES
ROWS_PER_WORKER = BATCH // NUM_WORKERS
LANES = 16
NUM_PASSES = 2
HALF = ROWS_PER_WORKER // NUM_PASSES
HGRP = HALF // LANES


def _body(a_hbm, b_hbm, emb_hbm, out_hbm,
          a_idx, b_idx, a_rows, b_rows, out_v, sem):
    wid = lax.axis_index("s") * NUM_CORES + lax.axis_index("c")
    base = wid * ROWS_PER_WORKER

    pltpu.sync_copy(a_hbm.at[pl.ds(base, ROWS_PER_WORKER)], a_idx)
    pltpu.sync_copy(b_hbm.at[pl.ds(base, ROWS_PER_WORKER)], b_idx)

    lane = lax.iota(jnp.int32, LANES)

    for p in range(NUM_PASSES):
        def fetch(i, carry):
            av = a_idx[pl.ds(p * HALF + i * LANES, LANES)]
            bv = b_idx[pl.ds(p * HALF + i * LANES, LANES)]
            aq = av // SUBLANES
            ar = av % SUBLANES
            bq = bv // SUBLANES
            br = bv % SUBLANES
            for j in range(LANES):
                q = 2 * i + j // SUBLANES
                r = j % SUBLANES
                pltpu.async_copy(emb_hbm.at[aq[j], ar[j]],
                                 a_rows.at[q, r], sem)
                pltpu.async_copy(emb_hbm.at[bq[j], br[j]],
                                 b_rows.at[q, r], sem)
            return carry

        lax.fori_loop(0, HALF // LANES, fetch, 0)
        pltpu.make_async_copy(emb_hbm.at[pl.ds(0, HALF // SUBLANES)],
                              a_rows, sem).wait()
        pltpu.make_async_copy(emb_hbm.at[pl.ds(0, HALF // SUBLANES)],
                              b_rows, sem).wait()

        def group(g, carry):
            sums = jnp.zeros((LANES,), jnp.float32)
            for r16 in range(LANES):
                row = g * LANES + r16
                q = row // SUBLANES
                r = row % SUBLANES
                acc = jnp.zeros((LANES,), jnp.float32)
                for k in range(EMBED_DIM // LANES):
                    sl = pl.ds(k * LANES, LANES)
                    acc = acc + a_rows[q, r, sl] * b_rows[q, r, sl]
                sums = jnp.where(lane == r16, jnp.sum(acc), sums)
            y = 1.0 / (1.0 + jnp.exp(-sums))
            out_v[pl.ds(p * HALF + g * LANES, LANES)] = y
            return carry

        lax.fori_loop(0, HGRP, group, 0)

    pltpu.sync_copy(out_v, out_hbm.at[pl.ds(base, ROWS_PER_WORKER)])


@jax.jit
def _cooc_dssm(a_nid, b_nid, nid_emb):
    emb3 = nid_emb.reshape(TILES, SUBLANES, EMBED_DIM)
    mesh = plsc.VectorSubcoreMesh(core_axis_name="c", subcore_axis_name="s")
    kern = functools.partial(
        pl.kernel,
        mesh=mesh,
        out_type=jax.ShapeDtypeStruct((BATCH,), jnp.float32),
        scratch_types=[
            pltpu.VMEM((ROWS_PER_WORKER,), jnp.int32),
            pltpu.VMEM((ROWS_PER_WORKER,), jnp.int32),
            pltpu.VMEM((HALF // SUBLANES, SUBLANES, EMBED_DIM), jnp.float32),
            pltpu.VMEM((HALF // SUBLANES, SUBLANES, EMBED_DIM), jnp.float32),
            pltpu.VMEM((ROWS_PER_WORKER,), jnp.float32),
            pltpu.SemaphoreType.DMA,
        ],
        compiler_params=pltpu.CompilerParams(
            needs_layout_passes=False, use_tc_tiling_on_sc=True),
    )(_body)
    return kern(a_nid, b_nid, emb3)


def kernel(a_nid, b_nid, nid_emb):
    return _cooc_dssm(a_nid.astype(jnp.int32), b_nid.astype(jnp.int32),
                      nid_emb)
